# Initial kernel scaffold; baseline (speedup 1.0000x reference)
import jax
import jax.numpy as jnp
from jax.experimental import pallas as pl


def kernel(x, edge_index, edge_label_index, Wl1, bl1, Wr1, Wl2, bl2, Wr2, Wl3, bl3, Wr3, Wl4, bl4, Wr4, Wd1, bd1, Wd2, bd2):
    raise NotImplementedError("write your pallas kernel here")


# speedup vs baseline: 3.9871x; 3.9871x over previous
import functools

import jax
import jax.numpy as jnp
from jax import lax
from jax.experimental import pallas as pl
from jax.experimental.pallas import tpu as pltpu
from jax.experimental.pallas import tpu_sc as plsc

N = 10000
H = 128
E = 320000
EL = 100000

NC = 2
NS = 16
NW = NC * NS

CHUNK = 80
NP_ = 10240
ROWS_PER_TILE = NP_ // NS
ZROWS = 128
E_PER_TILE = E // NW
E_CHUNKS = E_PER_TILE // CHUNK
EL_CHUNKS = EL // CHUNK
EL_ROUNDS = (EL_CHUNKS + NW - 1) // NW

_mesh = plsc.VectorSubcoreMesh(core_axis_name="c", subcore_axis_name="s",
                               num_cores=NC, num_subcores=NS)


def _zero_rows(buf, nrows, ncols):
    def body(r, _):
        for j in range(ncols // 16):
            buf[r, pl.ds(j * 16, 16)] = jnp.zeros((16,), jnp.float32)
        return 0
    lax.fori_loop(0, nrows, body, 0)


def _agg_kernel_body(u_hbm, src_hbm, dst_hbm, s_out,
                     src_v, dst_v, rows_v, stage_v, sem, acc):
    sid = lax.axis_index("s")
    cid = lax.axis_index("c")
    wid = sid * NC + cid

    _zero_rows(stage_v, ZROWS, H)
    for t in range(ROWS_PER_TILE // ZROWS):
        pltpu.sync_copy(stage_v,
                        acc.at[pl.ds(sid * ROWS_PER_TILE + t * ZROWS, ZROWS)])
    plsc.subcore_barrier()

    def body(i, _):
        base = wid * E_PER_TILE + i * CHUNK
        pltpu.sync_copy(src_hbm.at[pl.ds(base, CHUNK)], src_v)
        pltpu.sync_copy(dst_hbm.at[pl.ds(base, CHUNK)], dst_v)
        pltpu.async_copy(u_hbm.at[src_v], rows_v, sem).wait()
        pltpu.sync_copy(rows_v, acc.at[dst_v], add=True)
        return 0
    lax.fori_loop(0, E_CHUNKS, body, 0)
    plsc.subcore_barrier()

    for t in range(ROWS_PER_TILE // ZROWS):
        r0 = sid * ROWS_PER_TILE + t * ZROWS
        pltpu.sync_copy(acc.at[pl.ds(r0, ZROWS)], stage_v)
        pltpu.sync_copy(stage_v, s_out.at[cid].at[pl.ds(r0, ZROWS)])


_agg = pl.kernel(
    _agg_kernel_body,
    out_type=jax.ShapeDtypeStruct((NC, NP_, H), jnp.float32),
    mesh=_mesh,
    scratch_types=[
        pltpu.VMEM((CHUNK,), jnp.int32),
        pltpu.VMEM((CHUNK,), jnp.int32),
        pltpu.VMEM((CHUNK, H), jnp.float32),
        pltpu.VMEM((ZROWS, H), jnp.float32),
        pltpu.SemaphoreType.DMA,
        pltpu.VMEM_SHARED((NP_, H), jnp.float32),
    ],
)


def _deg_kernel_body(dst_hbm, dg_out, dst_v, ones_v, stage_v, dacc):
    sid = lax.axis_index("s")
    cid = lax.axis_index("c")
    wid = sid * NC + cid

    _zero_rows(stage_v, ZROWS, H)
    for t in range(ROWS_PER_TILE // ZROWS):
        pltpu.sync_copy(stage_v,
                        dacc.at[pl.ds(sid * ROWS_PER_TILE + t * ZROWS, ZROWS)])

    def ones_body(r, _):
        for j in range(H // 16):
            ones_v[r, pl.ds(j * 16, 16)] = jnp.ones((16,), jnp.float32)
        return 0
    lax.fori_loop(0, CHUNK, ones_body, 0)
    plsc.subcore_barrier()

    def body(i, _):
        base = wid * E_PER_TILE + i * CHUNK
        pltpu.sync_copy(dst_hbm.at[pl.ds(base, CHUNK)], dst_v)
        pltpu.sync_copy(ones_v, dacc.at[dst_v], add=True)
        return 0
    lax.fori_loop(0, E_CHUNKS, body, 0)
    plsc.subcore_barrier()

    for t in range(ROWS_PER_TILE // ZROWS):
        r0 = sid * ROWS_PER_TILE + t * ZROWS
        pltpu.sync_copy(dacc.at[pl.ds(r0, ZROWS)], stage_v)
        pltpu.sync_copy(stage_v, dg_out.at[cid].at[pl.ds(r0, ZROWS)])


_deg = pl.kernel(
    _deg_kernel_body,
    out_type=jax.ShapeDtypeStruct((NC, NP_, H), jnp.float32),
    mesh=_mesh,
    scratch_types=[
        pltpu.VMEM((CHUNK,), jnp.int32),
        pltpu.VMEM((CHUNK, H), jnp.float32),
        pltpu.VMEM((ZROWS, H), jnp.float32),
        pltpu.VMEM_SHARED((NP_, H), jnp.float32),
    ],
)


def _dec_gather_body(p_hbm, q_hbm, row_hbm, col_hbm, g_out,
                     ri_v, ci_v, pr_v, qr_v, g_v, sem):
    sid = lax.axis_index("s")
    cid = lax.axis_index("c")
    wid = sid * NC + cid

    def round_body(i, _):
        c = i * NW + wid

        @pl.when(c < EL_CHUNKS)
        def _():
            base = c * CHUNK
            pltpu.sync_copy(row_hbm.at[pl.ds(base, CHUNK)], ri_v)
            pltpu.sync_copy(col_hbm.at[pl.ds(base, CHUNK)], ci_v)
            pltpu.async_copy(p_hbm.at[ri_v], pr_v, sem).wait()
            pltpu.async_copy(q_hbm.at[ci_v], qr_v, sem).wait()

            def edge_body(e, _):
                for j in range(H // 16):
                    g_v[e, pl.ds(j * 16, 16)] = (
                        pr_v[e, pl.ds(j * 16, 16)]
                        + qr_v[e, pl.ds(j * 16, 16)])
                return 0
            lax.fori_loop(0, CHUNK, edge_body, 0)
            pltpu.sync_copy(g_v, g_out.at[pl.ds(base, CHUNK)])
        return 0
    lax.fori_loop(0, EL_ROUNDS, round_body, 0)


_dec_gather = pl.kernel(
    _dec_gather_body,
    out_type=jax.ShapeDtypeStruct((EL, H), jnp.float32),
    mesh=_mesh,
    scratch_types=[
        pltpu.VMEM((CHUNK,), jnp.int32),
        pltpu.VMEM((CHUNK,), jnp.int32),
        pltpu.VMEM((CHUNK, H), jnp.float32),
        pltpu.VMEM((CHUNK, H), jnp.float32),
        pltpu.VMEM((CHUNK, H), jnp.float32),
        pltpu.SemaphoreType.DMA,
    ],
)


_RB = 2000


def _tc_layer_body(relu, s_ref, dg_ref, z_ref, bl_ref, wl_ref, wr_ref,
                   zo_ref):
    deg = dg_ref[0, :, 0:1] + dg_ref[1, :, 0:1]
    agg = (s_ref[0] + s_ref[1]) / jnp.maximum(deg, 1.0)
    v = (jnp.dot(agg, wl_ref[...], preferred_element_type=jnp.float32)
         + bl_ref[...]
         + jnp.dot(z_ref[...], wr_ref[...], preferred_element_type=jnp.float32))
    if relu:
        v = jnp.maximum(v, 0.0)
    zo_ref[...] = v


def _make_tc_layer(relu):
    return pl.pallas_call(
        functools.partial(_tc_layer_body, relu),
        grid=(N // _RB,),
        in_specs=[
            pl.BlockSpec((NC, _RB, H), lambda i: (0, i, 0)),
            pl.BlockSpec((NC, _RB, H), lambda i: (0, i, 0)),
            pl.BlockSpec((_RB, H), lambda i: (i, 0)),
            pl.BlockSpec((H,), lambda i: (0,)),
            pl.BlockSpec((H, H), lambda i: (0, 0)),
            pl.BlockSpec((H, H), lambda i: (0, 0)),
        ],
        out_specs=pl.BlockSpec((_RB, H), lambda i: (i, 0)),
        out_shape=jax.ShapeDtypeStruct((N, H), jnp.float32),
    )


_tc_layer_relu = _make_tc_layer(True)


def _tc_last_body(s_ref, dg_ref, z_ref, bl_ref, wl_ref, wr_ref,
                  wa_ref, wb_ref, bd1_ref, p_ref, q_ref):
    deg = dg_ref[0, :, 0:1] + dg_ref[1, :, 0:1]
    agg = (s_ref[0] + s_ref[1]) / jnp.maximum(deg, 1.0)
    z5 = (jnp.dot(agg, wl_ref[...], preferred_element_type=jnp.float32)
          + bl_ref[...]
          + jnp.dot(z_ref[...], wr_ref[...], preferred_element_type=jnp.float32))
    p_ref[...] = (jnp.dot(z5, wa_ref[...], preferred_element_type=jnp.float32)
                  + bd1_ref[...])
    q_ref[...] = jnp.dot(z5, wb_ref[...], preferred_element_type=jnp.float32)


_tc_last = pl.pallas_call(
    _tc_last_body,
    grid=(N // _RB,),
    in_specs=[
        pl.BlockSpec((NC, _RB, H), lambda i: (0, i, 0)),
        pl.BlockSpec((NC, _RB, H), lambda i: (0, i, 0)),
        pl.BlockSpec((_RB, H), lambda i: (i, 0)),
        pl.BlockSpec((H,), lambda i: (0,)),
        pl.BlockSpec((H, H), lambda i: (0, 0)),
        pl.BlockSpec((H, H), lambda i: (0, 0)),
        pl.BlockSpec((H, H), lambda i: (0, 0)),
        pl.BlockSpec((H, H), lambda i: (0, 0)),
        pl.BlockSpec((H,), lambda i: (0,)),
    ],
    out_specs=[
        pl.BlockSpec((_RB, H), lambda i: (i, 0)),
        pl.BlockSpec((_RB, H), lambda i: (i, 0)),
    ],
    out_shape=[
        jax.ShapeDtypeStruct((N, H), jnp.float32),
        jax.ShapeDtypeStruct((N, H), jnp.float32),
    ],
)


_DB = 8


def _tc_dec_body(g_ref, w2_ref, b2_ref, o_ref):
    h = jnp.maximum(g_ref[...], 0.0)
    o_ref[...] = jnp.sum(h * w2_ref[...], axis=-1) + b2_ref[...]


_tc_dec = pl.pallas_call(
    _tc_dec_body,
    grid=(10,),
    in_specs=[
        pl.BlockSpec((_DB, 1250, H), lambda i: (i, 0, 0)),
        pl.BlockSpec((H,), lambda i: (0,)),
        pl.BlockSpec((_DB, 1250), lambda i: (i, 0)),
    ],
    out_specs=pl.BlockSpec((_DB, 1250), lambda i: (i, 0)),
    out_shape=jax.ShapeDtypeStruct((10 * _DB, 1250), jnp.float32),
)


def kernel(x, edge_index, edge_label_index,
           Wl1, bl1, Wr1, Wl2, bl2, Wr2, Wl3, bl3, Wr3, Wl4, bl4, Wr4,
           Wd1, bd1, Wd2, bd2):
    src = edge_index[0]
    dst = edge_index[1]
    row = edge_label_index[0]
    col = edge_label_index[1]

    dg = _deg(dst)
    s = _agg(x, src, dst)
    z = _tc_layer_relu(s, dg, x, bl1, Wl1, Wr1)
    s = _agg(z, src, dst)
    z = _tc_layer_relu(s, dg, z, bl2, Wl2, Wr2)
    s = _agg(z, src, dst)
    z = _tc_layer_relu(s, dg, z, bl3, Wl3, Wr3)
    s = _agg(z, src, dst)
    p, q = _tc_last(s, dg, z, bl4, Wl4, Wr4, Wd1[:H], Wd1[H:], bd1)

    g = _dec_gather(p, q, row, col)
    g3 = g.reshape(10 * _DB, 1250, H)
    b2 = jnp.broadcast_to(bd2, (10 * _DB, 1250))
    o2 = _tc_dec(g3, Wd2[:, 0], b2)
    return o2.reshape(-1)
